# baseline (device time: 104182 ns/iter reference)
import jax
import jax.numpy as jnp
from jax import lax
from jax.experimental import pallas as pl
from jax.experimental.pallas import tpu as pltpu

N_DEV = 8


def kernel(x, w_mat):
    k_full, k_shard = x.shape
    _, n = w_mat.shape
    m_blk = k_full // N_DEV

    def body(x_ref, w_ref, out_ref, xg_ref, amax_tx_ref, amax_rx_ref,
             send_sems, recv_sems, amax_send_sems, amax_recv_sems):
        my = lax.axis_index("i")

        barrier_sem = pltpu.get_barrier_semaphore()
        for d in range(1, N_DEV):
            peer = lax.rem(my + d, N_DEV)
            pl.semaphore_signal(barrier_sem, inc=1, device_id=(peer,),
                                device_id_type=pl.DeviceIdType.MESH)
        pl.semaphore_wait(barrier_sem, N_DEV - 1)

        sends = []
        for d in range(1, N_DEV):
            tgt = lax.rem(my + d, N_DEV)
            rdma = pltpu.make_async_remote_copy(
                src_ref=x_ref.at[pl.ds(tgt * m_blk, m_blk), :],
                dst_ref=xg_ref.at[:, pl.ds(my * k_shard, k_shard)],
                send_sem=send_sems.at[d - 1],
                recv_sem=recv_sems.at[d - 1],
                device_id=(tgt,),
                device_id_type=pl.DeviceIdType.MESH,
            )
            rdma.start()
            sends.append(rdma)

        xg_ref[:, pl.ds(my * k_shard, k_shard)] = x_ref[pl.ds(my * m_blk, m_blk), :]

        for d in range(1, N_DEV):
            src = lax.rem(my + (N_DEV - d), N_DEV)
            recv = pltpu.make_async_remote_copy(
                src_ref=x_ref.at[pl.ds(0, m_blk), :],
                dst_ref=xg_ref.at[:, pl.ds(src * k_shard, k_shard)],
                send_sem=send_sems.at[d - 1],
                recv_sem=recv_sems.at[d - 1],
                device_id=(my,),
                device_id_type=pl.DeviceIdType.MESH,
            )
            recv.wait_recv()
        for rdma in sends:
            rdma.wait_send()

        y = jnp.maximum(
            jnp.dot(xg_ref[...], w_ref[...], preferred_element_type=jnp.float32),
            0.0,
        )

        local_amax = jnp.max(y)
        amax_tx_ref[...] = jnp.full((8, 128), local_amax, jnp.float32)
        amax_rx_ref[pl.ds(my, 1)] = amax_tx_ref[...][None]
        amax_sends = []
        for d in range(1, N_DEV):
            tgt = lax.rem(my + d, N_DEV)
            rdma = pltpu.make_async_remote_copy(
                src_ref=amax_tx_ref,
                dst_ref=amax_rx_ref.at[my],
                send_sem=amax_send_sems.at[d - 1],
                recv_sem=amax_recv_sems.at[d - 1],
                device_id=(tgt,),
                device_id_type=pl.DeviceIdType.MESH,
            )
            rdma.start()
            amax_sends.append(rdma)
        for d in range(1, N_DEV):
            src = lax.rem(my + (N_DEV - d), N_DEV)
            recv = pltpu.make_async_remote_copy(
                src_ref=amax_tx_ref,
                dst_ref=amax_rx_ref.at[src],
                send_sem=amax_send_sems.at[d - 1],
                recv_sem=amax_recv_sems.at[d - 1],
                device_id=(my,),
                device_id_type=pl.DeviceIdType.MESH,
            )
            recv.wait_recv()
        for rdma in amax_sends:
            rdma.wait_send()

        scale = jnp.max(amax_rx_ref[...]) / 448.0
        q = (y / scale).astype(jnp.float8_e4m3fn)
        out_ref[...] = q.astype(jnp.float32) * scale

    return pl.pallas_call(
        body,
        out_shape=jax.ShapeDtypeStruct((m_blk, n), jnp.float32),
        in_specs=[
            pl.BlockSpec(memory_space=pltpu.VMEM),
            pl.BlockSpec(memory_space=pltpu.VMEM),
        ],
        out_specs=pl.BlockSpec(memory_space=pltpu.VMEM),
        scratch_shapes=[
            pltpu.VMEM((m_blk, k_full), jnp.float32),
            pltpu.VMEM((8, 128), jnp.float32),
            pltpu.VMEM((N_DEV, 8, 128), jnp.float32),
            pltpu.SemaphoreType.DMA((N_DEV - 1,)),
            pltpu.SemaphoreType.DMA((N_DEV - 1,)),
            pltpu.SemaphoreType.DMA((N_DEV - 1,)),
            pltpu.SemaphoreType.DMA((N_DEV - 1,)),
        ],
        compiler_params=pltpu.CompilerParams(
            collective_id=0,
            vmem_limit_bytes=100 * 1024 * 1024,
        ),
    )(x, w_mat)


# device time: 97186 ns/iter; 1.0720x vs baseline; 1.0720x over previous
import jax
import jax.numpy as jnp
from jax import lax
from jax.experimental import pallas as pl
from jax.experimental.pallas import tpu as pltpu

N_DEV = 8


def kernel(x, w_mat):
    k_full, k_shard = x.shape
    _, n = w_mat.shape
    m_blk = k_full // N_DEV

    def body(x_ref, w_ref, out_ref, xg_ref, amax_tx_ref, amax_rx_ref,
             send_sems, recv_sems, amax_send_sems, amax_recv_sems):
        my = lax.axis_index("i")

        barrier_sem = pltpu.get_barrier_semaphore()
        for d in range(1, N_DEV):
            peer = lax.rem(my + d, N_DEV)
            pl.semaphore_signal(barrier_sem, inc=1, device_id=(peer,),
                                device_id_type=pl.DeviceIdType.MESH)
        pl.semaphore_wait(barrier_sem, N_DEV - 1)

        sends = []
        for d in range(1, N_DEV):
            tgt = lax.rem(my + d, N_DEV)
            rdma = pltpu.make_async_remote_copy(
                src_ref=x_ref.at[pl.ds(tgt * m_blk, m_blk), :],
                dst_ref=xg_ref.at[my],
                send_sem=send_sems.at[d - 1],
                recv_sem=recv_sems.at[d - 1],
                device_id=(tgt,),
                device_id_type=pl.DeviceIdType.MESH,
            )
            rdma.start()
            sends.append(rdma)

        out_ref[...] = jnp.dot(
            x_ref[pl.ds(my * m_blk, m_blk), :],
            w_ref[pl.ds(my * k_shard, k_shard), :],
            preferred_element_type=jnp.float32,
        )

        for d in range(1, N_DEV):
            src = lax.rem(my + (N_DEV - d), N_DEV)
            recv = pltpu.make_async_remote_copy(
                src_ref=x_ref.at[pl.ds(0, m_blk), :],
                dst_ref=xg_ref.at[src],
                send_sem=send_sems.at[d - 1],
                recv_sem=recv_sems.at[d - 1],
                device_id=(my,),
                device_id_type=pl.DeviceIdType.MESH,
            )
            recv.wait_recv()
            out_ref[...] += jnp.dot(
                xg_ref[src],
                w_ref[pl.ds(src * k_shard, k_shard), :],
                preferred_element_type=jnp.float32,
            )
        y = jnp.maximum(out_ref[...], 0.0)

        local_amax = jnp.max(y)
        out_ref[...] = y

        amax_tx_ref[...] = jnp.full((8, 128), local_amax, jnp.float32)
        amax_rx_ref[pl.ds(my, 1)] = amax_tx_ref[...][None]
        amax_sends = []
        for d in range(1, N_DEV):
            tgt = lax.rem(my + d, N_DEV)
            rdma = pltpu.make_async_remote_copy(
                src_ref=amax_tx_ref,
                dst_ref=amax_rx_ref.at[my],
                send_sem=amax_send_sems.at[d - 1],
                recv_sem=amax_recv_sems.at[d - 1],
                device_id=(tgt,),
                device_id_type=pl.DeviceIdType.MESH,
            )
            rdma.start()
            amax_sends.append(rdma)
        for d in range(1, N_DEV):
            src = lax.rem(my + (N_DEV - d), N_DEV)
            recv = pltpu.make_async_remote_copy(
                src_ref=amax_tx_ref,
                dst_ref=amax_rx_ref.at[src],
                send_sem=amax_send_sems.at[d - 1],
                recv_sem=amax_recv_sems.at[d - 1],
                device_id=(my,),
                device_id_type=pl.DeviceIdType.MESH,
            )
            recv.wait_recv()
        for rdma in amax_sends:
            rdma.wait_send()
        for rdma in sends:
            rdma.wait_send()

        scale = jnp.max(amax_rx_ref[...]) / 448.0
        q = (out_ref[...] / scale).astype(jnp.float8_e4m3fn)
        out_ref[...] = q.astype(jnp.float32) * scale

    return pl.pallas_call(
        body,
        out_shape=jax.ShapeDtypeStruct((m_blk, n), jnp.float32),
        in_specs=[
            pl.BlockSpec(memory_space=pltpu.VMEM),
            pl.BlockSpec(memory_space=pltpu.VMEM),
        ],
        out_specs=pl.BlockSpec(memory_space=pltpu.VMEM),
        scratch_shapes=[
            pltpu.VMEM((N_DEV, m_blk, k_shard), jnp.float32),
            pltpu.VMEM((8, 128), jnp.float32),
            pltpu.VMEM((N_DEV, 8, 128), jnp.float32),
            pltpu.SemaphoreType.DMA((N_DEV - 1,)),
            pltpu.SemaphoreType.DMA((N_DEV - 1,)),
            pltpu.SemaphoreType.DMA((N_DEV - 1,)),
            pltpu.SemaphoreType.DMA((N_DEV - 1,)),
        ],
        compiler_params=pltpu.CompilerParams(
            collective_id=0,
            vmem_limit_bytes=100 * 1024 * 1024,
        ),
    )(x, w_mat)
